# Initial kernel scaffold; baseline (speedup 1.0000x reference)
import jax
import jax.numpy as jnp
from jax.experimental import pallas as pl


def kernel(x, edge_index, W1, b1, W2, b2, M1, mb1, M2, mb2):
    raise NotImplementedError("write your pallas kernel here")


# speedup vs baseline: 12.8959x; 12.8959x over previous
import functools

import jax
import jax.numpy as jnp
from jax import lax
from jax.experimental import pallas as pl
from jax.experimental.pallas import tpu as pltpu
from jax.experimental.pallas import tpu_sc as plsc

N = 10000
E = 320000
NP = 10240
EP = 327680
NC = 2
NS = 16
NW = NC * NS
ET = EP // NW
IROWS = ET // 128
CB = 1024
CROWS = CB // 128
NCHUNK = ET // CB
NZ = NP // NS

_mesh = plsc.VectorSubcoreMesh(core_axis_name="c", subcore_axis_name="s")


@functools.partial(
    pl.kernel,
    out_type=jax.ShapeDtypeStruct((NC, NP), jnp.float32),
    mesh=_mesh,
    compiler_params=pltpu.CompilerParams(use_tc_tiling_on_sc=False),
    scratch_types=[
        pltpu.VMEM((IROWS, 128), jnp.int32),
        pltpu.VMEM((128,), jnp.float32),
        pltpu.VMEM_SHARED((NP,), jnp.float32),
        pltpu.SemaphoreType.DMA,
    ],
)
def _deg_kernel(dst_hbm, zeros_hbm, out_hbm, idx_v, ones_v, acc, sem):
    c = lax.axis_index("c")
    s = lax.axis_index("s")
    w = c * NS + s
    pltpu.sync_copy(dst_hbm.at[w], idx_v)
    for j in range(8):
        ones_v[pl.ds(j * 16, 16)] = jnp.ones((16,), jnp.float32)
    pltpu.sync_copy(zeros_hbm.at[pl.ds(s * NZ, NZ)], acc.at[pl.ds(s * NZ, NZ)])
    plsc.subcore_barrier()

    def chunk(g, carry):
        hs = []
        for j in range(CROWS):
            hs.append(pltpu.async_copy(
                ones_v, acc.at[idx_v.at[g * CROWS + j]], sem, add=True))
        for h in hs:
            h.wait()
        return carry

    lax.fori_loop(0, NCHUNK, chunk, 0)
    plsc.subcore_barrier()
    pltpu.sync_copy(acc.at[pl.ds(s * NZ, NZ)], out_hbm.at[c, pl.ds(s * NZ, NZ)])


def _make_agg(D):
    @functools.partial(
        pl.kernel,
        out_type=jax.ShapeDtypeStruct((NC, NP, D), jnp.float32),
        mesh=_mesh,
        compiler_params=pltpu.CompilerParams(use_tc_tiling_on_sc=False),
        scratch_types=[
            pltpu.VMEM((IROWS, 128), jnp.int32),
            pltpu.VMEM((IROWS, 128), jnp.int32),
            pltpu.VMEM((CB, D), jnp.float32),
            pltpu.VMEM_SHARED((NP, D), jnp.float32),
            pltpu.SemaphoreType.DMA,
            pltpu.SemaphoreType.DMA,
        ],
    )
    def _agg(hs_hbm, src_hbm, dst_hbm, zeros_hbm, out_hbm,
             isv, idv, rows, acc, gsem, ssem):
        c = lax.axis_index("c")
        s = lax.axis_index("s")
        w = c * NS + s
        pltpu.sync_copy(src_hbm.at[w], isv)
        pltpu.sync_copy(dst_hbm.at[w], idv)
        pltpu.sync_copy(zeros_hbm.at[pl.ds(s * NZ, NZ)],
                        acc.at[pl.ds(s * NZ, NZ)])
        plsc.subcore_barrier()

        def chunk(g, carry):
            hs = []
            for j in range(CROWS):
                hs.append(pltpu.async_copy(
                    hs_hbm.at[isv.at[g * CROWS + j]],
                    rows.at[pl.ds(j * 128, 128)], gsem))
            for h in hs:
                h.wait()
            sc = []
            for j in range(CROWS):
                sc.append(pltpu.async_copy(
                    rows.at[pl.ds(j * 128, 128)],
                    acc.at[idv.at[g * CROWS + j]], ssem, add=True))
            for h in sc:
                h.wait()
            return carry

        lax.fori_loop(0, NCHUNK, chunk, 0)
        plsc.subcore_barrier()
        pltpu.sync_copy(acc.at[pl.ds(s * NZ, NZ)],
                        out_hbm.at[c, pl.ds(s * NZ, NZ)])

    return _agg


_agg32 = _make_agg(32)
_agg16 = _make_agg(16)


@functools.partial(
    pl.kernel,
    out_type=(jax.ShapeDtypeStruct((EP, 16), jnp.float32),
              jax.ShapeDtypeStruct((EP, 16), jnp.float32)),
    mesh=_mesh,
    compiler_params=pltpu.CompilerParams(use_tc_tiling_on_sc=False),
    scratch_types=[
        pltpu.VMEM((IROWS, 128), jnp.int32),
        pltpu.VMEM((IROWS, 128), jnp.int32),
        pltpu.VMEM((CB, 16), jnp.float32),
        pltpu.SemaphoreType.DMA,
    ],
)
def _edge_gather(h_hbm, src_hbm, dst_hbm, gs_hbm, gd_hbm, isv, idv, rows, sem):
    c = lax.axis_index("c")
    s = lax.axis_index("s")
    w = c * NS + s
    pltpu.sync_copy(src_hbm.at[w], isv)
    pltpu.sync_copy(dst_hbm.at[w], idv)
    base = w * ET

    def chunk(g, carry):
        off = base + g * CB
        hs = []
        for j in range(CROWS):
            hs.append(pltpu.async_copy(
                h_hbm.at[isv.at[g * CROWS + j]],
                rows.at[pl.ds(j * 128, 128)], sem))
        for h in hs:
            h.wait()
        pltpu.sync_copy(rows, gs_hbm.at[pl.ds(off, CB)])
        hd = []
        for j in range(CROWS):
            hd.append(pltpu.async_copy(
                h_hbm.at[idv.at[g * CROWS + j]],
                rows.at[pl.ds(j * 128, 128)], sem))
        for h in hd:
            h.wait()
        pltpu.sync_copy(rows, gd_hbm.at[pl.ds(off, CB)])
        return carry

    lax.fori_loop(0, NCHUNK, chunk, 0)


def _tc1_body(deg_ref, x_ref, w_ref, o_ref):
    dinv = lax.rsqrt(deg_ref[0, :] + deg_ref[1, :] + 1.0)
    h = jnp.dot(x_ref[...], w_ref[...], preferred_element_type=jnp.float32)
    o_ref[...] = h * dinv[:, None]


def _tc2_body(deg_ref, s1_ref, h1s_ref, w2_ref, b1_ref, o_ref):
    dinv = lax.rsqrt(deg_ref[0, :] + deg_ref[1, :] + 1.0)[:, None]
    pre = dinv * (s1_ref[0] + s1_ref[1] + h1s_ref[...]) + b1_ref[...]
    h1r = jnp.maximum(pre, 0.0)
    h2 = jnp.dot(h1r, w2_ref[...], preferred_element_type=jnp.float32)
    o_ref[...] = h2 * dinv


def _tc3_body(deg_ref, s2_ref, h2s_ref, b2_ref, o_ref):
    dinv = lax.rsqrt(deg_ref[0, :] + deg_ref[1, :] + 1.0)[:, None]
    o_ref[...] = dinv * (s2_ref[0] + s2_ref[1] + h2s_ref[...]) + b2_ref[...]


BE = 8192


def _mlp_body(mb2_ref, gs_ref, gd_ref, m1s_ref, m1d_ref, mb1_ref, m2_ref,
              o_ref):
    hid = jnp.dot(gs_ref[...], m1s_ref[...], preferred_element_type=jnp.float32)
    hid = hid + jnp.dot(gd_ref[...], m1d_ref[...],
                        preferred_element_type=jnp.float32)
    hid = jnp.maximum(hid + mb1_ref[...], 0.0)
    o_ref[...] = jnp.sum(hid * m2_ref[...], axis=1) + mb2_ref[0]


def kernel(x, edge_index, W1, b1, W2, b2, M1, mb1, M2, mb2):
    src = edge_index[0].astype(jnp.int32)
    dst = edge_index[1].astype(jnp.int32)
    pad = jnp.full((EP - E,), N, jnp.int32)
    srcR = jnp.concatenate([src, pad]).reshape(NW, IROWS, 128)
    dstR = jnp.concatenate([dst, pad]).reshape(NW, IROWS, 128)
    xp = jnp.pad(x, ((0, NP - N), (0, 0)))
    W1p = jnp.pad(W1, ((0, 0), (0, 12)))
    b1p = jnp.pad(b1, (0, 12)).reshape(1, 32)
    W2p = jnp.pad(W2, ((0, 12), (0, 0)))
    b2r = b2.reshape(1, 16)
    z1 = jnp.zeros((NP,), jnp.float32)
    z32 = jnp.zeros((NP, 32), jnp.float32)
    z16 = jnp.zeros((NP, 16), jnp.float32)

    deg2 = _deg_kernel(dstR, z1)

    h1s = pl.pallas_call(
        _tc1_body,
        out_shape=jax.ShapeDtypeStruct((NP, 32), jnp.float32),
    )(deg2, xp, W1p)

    S1 = _agg32(h1s, srcR, dstR, z32)

    h2s = pl.pallas_call(
        _tc2_body,
        out_shape=jax.ShapeDtypeStruct((NP, 16), jnp.float32),
    )(deg2, S1, h1s, W2p, b1p)

    S2 = _agg16(h2s, srcR, dstR, z16)

    h = pl.pallas_call(
        _tc3_body,
        out_shape=jax.ShapeDtypeStruct((NP, 16), jnp.float32),
    )(deg2, S2, h2s, b2r)

    gs, gd = _edge_gather(h, srcR, dstR)

    logits = pl.pallas_call(
        _mlp_body,
        grid=(EP // BE,),
        in_specs=[
            pl.BlockSpec(memory_space=pltpu.SMEM),
            pl.BlockSpec((BE, 16), lambda i: (i, 0)),
            pl.BlockSpec((BE, 16), lambda i: (i, 0)),
            pl.BlockSpec((16, 64), lambda i: (0, 0)),
            pl.BlockSpec((16, 64), lambda i: (0, 0)),
            pl.BlockSpec((1, 64), lambda i: (0, 0)),
            pl.BlockSpec((1, 64), lambda i: (0, 0)),
        ],
        out_specs=pl.BlockSpec((BE,), lambda i: (i,)),
        out_shape=jax.ShapeDtypeStruct((EP,), jnp.float32),
    )(mb2, gs, gd, M1[:16], M1[16:], mb1.reshape(1, 64),
      M2.reshape(1, 64))

    return logits[:E].reshape(E, 1)
